# baseline (speedup 1.0000x reference)
import functools
import math

import jax
import jax.numpy as jnp
import numpy as np
from jax import lax
from jax.experimental import pallas as pl
from jax.experimental.pallas import tpu as pltpu
from jax.experimental.pallas import tpu_sc as plsc

_H = 32
_W = 32
_C = 96
_C8 = 8 * _C
_P = _H * _W


def _build_maps():
    center = (_H - 1) / 2.0
    learnable = []
    for i in range(_H):
        for j in range(_W):
            y = center - i
            x = j - center
            if x == 0 and y == 0:
                learnable.append((i, j))
            else:
                ang = math.atan2(y, x)
                if ang < 0:
                    ang += 2 * math.pi
                if 0 <= ang <= math.pi / 4 + 1e-06:
                    learnable.append((i, j))
    src = -np.ones(_P, dtype=np.int64)
    rot = np.zeros(_P, dtype=np.int64)
    for idx, (i, j) in enumerate(learnable):
        for k in range(8):
            y = center - i
            x = j - center
            theta = k * math.pi / 4
            cos_t = math.cos(theta)
            sin_t = math.sin(theta)
            x_new = cos_t * x - sin_t * y
            y_new = sin_t * x + cos_t * y
            i_r = int(round(center - y_new))
            j_r = int(round(center + x_new))
            i_r = max(0, min(_H - 1, i_r))
            j_r = max(0, min(_W - 1, j_r))
            p = i_r * _W + j_r
            src[p] = idx
            rot[p] = k
    mask = src >= 0
    src = np.where(mask, src, 0)
    return len(learnable), src, rot, mask


_NL, _SRC, _ROT, _MASK = _build_maps()

_SC_NC = 2
_SC_NS = 16
_SC_NW = _SC_NC * _SC_NS
_GB = 8448
_BPW = _GB // _SC_NW
_NCHK = 3
_CHK = _BPW // _NCHK


def _build_gather_idx():
    zero_row = 1 + 8 * _NL
    idx = np.full((_GB,), zero_row, dtype=np.int32)
    idx[0:8] = 0
    n = np.arange(8)
    perm = (n[None, :] - _ROT[:, None]) % 8
    flat = 1 + 8 * _SRC[:, None] + perm
    flat = np.where(_MASK[:, None], flat, zero_row)
    idx[8 : 8 + 8 * _P] = flat.reshape(-1)
    return idx.reshape(_SC_NW, _NCHK, _CHK)


_GIDX = _build_gather_idx()


def _sc_gather_body(table_hbm, idx_hbm, out_hbm, idx_v, rows_v, sem):
    wid = lax.axis_index("s") * _SC_NC + lax.axis_index("c")
    base = wid * _BPW
    pltpu.sync_copy(idx_hbm.at[wid], idx_v)
    copies = [
        pltpu.async_copy(
            table_hbm.at[idx_v.at[j]], rows_v.at[pl.ds(j * _CHK, _CHK)], sem
        )
        for j in range(_NCHK)
    ]
    for c in copies:
        c.wait()
    pltpu.sync_copy(rows_v, out_hbm.at[pl.ds(base, _BPW)])


def _sc_gather(table, gidx):
    mesh = plsc.VectorSubcoreMesh(core_axis_name="c", subcore_axis_name="s")
    k = pl.kernel(
        _sc_gather_body,
        mesh=mesh,
        out_type=jax.ShapeDtypeStruct((_GB, 128), jnp.float32),
        scratch_types=[
            pltpu.VMEM((_NCHK, _CHK), jnp.int32),
            pltpu.VMEM((_BPW, 128), jnp.float32),
            pltpu.SemaphoreType.DMA,
        ],
    )
    return k(table, gidx)


_RBLK = 32


def _add_body(g_ref, x_ref, o_ref):
    o_ref[...] = x_ref[...] + g_ref[...][:, None, :]


@jax.jit
def kernel(x, pos_embed_learnable, cls_pos_eighth):
    B = x.shape[0]
    pe = pos_embed_learnable[0]
    cls = cls_pos_eighth[0]
    table = jnp.concatenate(
        [cls, pe.reshape(8 * _NL, _C), jnp.zeros((1, _C), jnp.float32)], axis=0
    )
    table = jnp.pad(table, ((0, 0), (0, 128 - _C)))
    flat_grid = _sc_gather(table, jnp.asarray(_GIDX))
    full_grid = flat_grid[:, :_C].reshape(_GB // 8, _C8)
    xt = jnp.transpose(x, (1, 0, 2))
    nblk = (1 + _P + _RBLK - 1) // _RBLK
    out_t = pl.pallas_call(
        _add_body,
        grid=(nblk,),
        in_specs=[
            pl.BlockSpec((_RBLK, _C8), lambda i: (i, 0)),
            pl.BlockSpec((_RBLK, B, _C8), lambda i: (i, 0, 0)),
        ],
        out_specs=pl.BlockSpec((_RBLK, B, _C8), lambda i: (i, 0, 0)),
        out_shape=jax.ShapeDtypeStruct(xt.shape, x.dtype),
    )(full_grid, xt)
    return jnp.transpose(out_t, (1, 0, 2))


# speedup vs baseline: 1.5961x; 1.5961x over previous
import math

import jax
import jax.numpy as jnp
import numpy as np
from jax.experimental import pallas as pl
from jax.experimental.pallas import tpu as pltpu

_H = 32
_W = 32
_C = 96
_C8 = 8 * _C
_P = _H * _W


def _build_maps():
    center = (_H - 1) / 2.0
    learnable = []
    for i in range(_H):
        for j in range(_W):
            y = center - i
            x = j - center
            if x == 0 and y == 0:
                learnable.append((i, j))
            else:
                ang = math.atan2(y, x)
                if ang < 0:
                    ang += 2 * math.pi
                if 0 <= ang <= math.pi / 4 + 1e-06:
                    learnable.append((i, j))
    src = -np.ones(_P, dtype=np.int64)
    rot = np.zeros(_P, dtype=np.int64)
    for idx, (i, j) in enumerate(learnable):
        for k in range(8):
            y = center - i
            x = j - center
            theta = k * math.pi / 4
            cos_t = math.cos(theta)
            sin_t = math.sin(theta)
            x_new = cos_t * x - sin_t * y
            y_new = sin_t * x + cos_t * y
            i_r = int(round(center - y_new))
            j_r = int(round(center + x_new))
            i_r = max(0, min(_H - 1, i_r))
            j_r = max(0, min(_W - 1, j_r))
            p = i_r * _W + j_r
            src[p] = idx
            rot[p] = k
    mask = src >= 0
    src = np.where(mask, src, 0)
    return len(learnable), src, rot, mask


_NL, _SRC, _ROT, _MASK = _build_maps()

_ONEHOT = np.zeros((_P, _NL), dtype=np.float32)
_ONEHOT[np.arange(_P), _SRC] = 1.0
_ROTMASKS = np.stack(
    [((_ROT == k) & _MASK).astype(np.float32) for k in range(8)], axis=0
)


def _grid_body(onehot_ref, masks_ref, pe_ref, cls_ref, g_ref):
    rows = jnp.dot(
        onehot_ref[...], pe_ref[...], preferred_element_type=jnp.float32
    )
    acc = jnp.zeros((_P, _C8), jnp.float32)
    for k in range(8):
        s = ((8 - k) % 8) * _C
        if s:
            rolled = jnp.concatenate([rows[:, s:], rows[:, :s]], axis=1)
        else:
            rolled = rows
        acc = acc + masks_ref[k, :][:, None] * rolled
    g_ref[0:1, :] = jnp.concatenate([cls_ref[...]] * 8, axis=1)
    g_ref[1:, :] = acc


_RBLK = 32


def _add_body(g_ref, x_ref, o_ref):
    o_ref[...] = x_ref[...] + g_ref[...][:, None, :]


@jax.jit
def kernel(x, pos_embed_learnable, cls_pos_eighth):
    B = x.shape[0]
    pe = pos_embed_learnable[0]
    cls = cls_pos_eighth[0]
    onehot = jnp.asarray(_ONEHOT)
    masks = jnp.asarray(_ROTMASKS)
    full_grid = pl.pallas_call(
        _grid_body,
        out_shape=jax.ShapeDtypeStruct((1 + _P, _C8), jnp.float32),
    )(onehot, masks, pe, cls)
    xt = jnp.transpose(x, (1, 0, 2))
    nblk = (1 + _P + _RBLK - 1) // _RBLK
    out_t = pl.pallas_call(
        _add_body,
        grid=(nblk,),
        in_specs=[
            pl.BlockSpec((_RBLK, _C8), lambda i: (i, 0)),
            pl.BlockSpec((_RBLK, B, _C8), lambda i: (i, 0, 0)),
        ],
        out_specs=pl.BlockSpec((_RBLK, B, _C8), lambda i: (i, 0, 0)),
        out_shape=jax.ShapeDtypeStruct(xt.shape, x.dtype),
    )(full_grid, xt)
    return jnp.transpose(out_t, (1, 0, 2))


# speedup vs baseline: 1.6406x; 1.0279x over previous
import math

import jax
import jax.numpy as jnp
import numpy as np
from jax.experimental import pallas as pl
from jax.experimental.pallas import tpu as pltpu

_H = 32
_W = 32
_C = 96
_C8 = 8 * _C
_P = _H * _W


def _build_maps():
    center = (_H - 1) / 2.0
    learnable = []
    for i in range(_H):
        for j in range(_W):
            y = center - i
            x = j - center
            if x == 0 and y == 0:
                learnable.append((i, j))
            else:
                ang = math.atan2(y, x)
                if ang < 0:
                    ang += 2 * math.pi
                if 0 <= ang <= math.pi / 4 + 1e-06:
                    learnable.append((i, j))
    src = -np.ones(_P, dtype=np.int64)
    rot = np.zeros(_P, dtype=np.int64)
    for idx, (i, j) in enumerate(learnable):
        for k in range(8):
            y = center - i
            x = j - center
            theta = k * math.pi / 4
            cos_t = math.cos(theta)
            sin_t = math.sin(theta)
            x_new = cos_t * x - sin_t * y
            y_new = sin_t * x + cos_t * y
            i_r = int(round(center - y_new))
            j_r = int(round(center + x_new))
            i_r = max(0, min(_H - 1, i_r))
            j_r = max(0, min(_W - 1, j_r))
            p = i_r * _W + j_r
            src[p] = idx
            rot[p] = k
    mask = src >= 0
    src = np.where(mask, src, 0)
    return len(learnable), src, rot, mask


_NL, _SRC, _ROT, _MASK = _build_maps()

_ONEHOT = np.zeros((_P, _NL), dtype=np.float32)
_ONEHOT[np.arange(_P), _SRC] = 1.0
_ROTMASKS = np.stack(
    [((_ROT == k) & _MASK).astype(np.float32) for k in range(8)], axis=0
)


_RBLK = 32


def _fused_body(onehot_ref, masks_ref, pe_ref, cls_ref, x_ref, o_ref, g_scr):
    i = pl.program_id(0)

    @pl.when(i == 0)
    def _build_grid():
        rows = jnp.dot(
            onehot_ref[...], pe_ref[...], preferred_element_type=jnp.float32
        )
        acc = jnp.zeros((_P, _C8), jnp.float32)
        for k in range(8):
            s = ((8 - k) % 8) * _C
            if s:
                rolled = jnp.concatenate([rows[:, s:], rows[:, :s]], axis=1)
            else:
                rolled = rows
            acc = acc + masks_ref[k, :][:, None] * rolled
        g_scr[0:1, :] = jnp.concatenate([cls_ref[...]] * 8, axis=1)
        g_scr[1 : 1 + _P, :] = acc

    g = g_scr[pl.ds(i * _RBLK, _RBLK), :]
    o_ref[...] = x_ref[...] + g[:, None, :]


@jax.jit
def kernel(x, pos_embed_learnable, cls_pos_eighth):
    B = x.shape[0]
    pe = pos_embed_learnable[0]
    cls = cls_pos_eighth[0]
    onehot = jnp.asarray(_ONEHOT)
    masks = jnp.asarray(_ROTMASKS)
    xt = jnp.transpose(x, (1, 0, 2))
    nblk = (1 + _P + _RBLK - 1) // _RBLK
    out_t = pl.pallas_call(
        _fused_body,
        grid=(nblk,),
        in_specs=[
            pl.BlockSpec((_P, _NL), lambda i: (0, 0)),
            pl.BlockSpec((8, _P), lambda i: (0, 0)),
            pl.BlockSpec((_NL, _C8), lambda i: (0, 0)),
            pl.BlockSpec((1, _C), lambda i: (0, 0)),
            pl.BlockSpec((_RBLK, B, _C8), lambda i: (i, 0, 0)),
        ],
        out_specs=pl.BlockSpec((_RBLK, B, _C8), lambda i: (i, 0, 0)),
        out_shape=jax.ShapeDtypeStruct(xt.shape, x.dtype),
        scratch_shapes=[
            pltpu.VMEM((nblk * _RBLK, _C8), jnp.float32),
        ],
    )(onehot, masks, pe, cls, xt)
    return jnp.transpose(out_t, (1, 0, 2))
